# Initial kernel scaffold; baseline (speedup 1.0000x reference)
import jax
import jax.numpy as jnp
from jax.experimental import pallas as pl


def kernel(feat_index, feat_value, emb_table, first_order_w, bias):
    raise NotImplementedError("write your pallas kernel here")


# speedup vs baseline: 1.1578x; 1.1578x over previous
import functools

import jax
import jax.numpy as jnp
from jax import lax
from jax.experimental import pallas as pl
from jax.experimental.pallas import tpu as pltpu
from jax.experimental.pallas import tpu_sc as plsc


def _build_fm(B, F, V, D):
  info = plsc.get_sparse_core_info()
  NC, NS, L = info.num_cores, info.num_subcores, info.num_lanes
  NW = NC * NS
  assert D == L and B % (L * NW) == 0
  RPW = B // NW
  NB = 64
  NCHUNK = RPW // NB
  IPC = NB * F
  assert IPC % 128 == 0
  NG = IPC // 128
  NGRP = NB // L

  mesh = plsc.VectorSubcoreMesh(core_axis_name="c", subcore_axis_name="s")

  @functools.partial(
      pl.kernel,
      mesh=mesh,
      compiler_params=pltpu.CompilerParams(
          needs_layout_passes=False, use_tc_tiling_on_sc=False),
      out_type=jax.ShapeDtypeStruct((B,), jnp.float32),
      scratch_types=[
          pltpu.VMEM((IPC,), jnp.int32),
          pltpu.VMEM((IPC,), jnp.float32),
          pltpu.VMEM((IPC,), jnp.float32),
          pltpu.VMEM((IPC, D), jnp.float32),
          pltpu.VMEM((L,), jnp.float32),
          pltpu.VMEM((RPW,), jnp.float32),
          pltpu.SemaphoreType.DMA,
      ],
  )
  def fm(fi_hbm, fv_hbm, emb_hbm, fo_hbm, bias_hbm, out_hbm,
         idx_v, fv_v, fo_v, rows_v, bias_v, out_v, sem):
    wid = lax.axis_index("s") * NC + lax.axis_index("c")
    pltpu.sync_copy(bias_hbm, bias_v)
    iota = lax.iota(jnp.int32, L)

    def chunk_body(c, carry):
      gchunk = wid * NCHUNK + c
      pltpu.sync_copy(fi_hbm.at[pl.ds(gchunk * IPC, IPC)], idx_v)
      pltpu.sync_copy(fv_hbm.at[pl.ds(gchunk * IPC, IPC)], fv_v)
      copies = []
      for j in range(NG):
        copies.append(pltpu.async_copy(
            emb_hbm.at[idx_v.at[pl.ds(j * 128, 128)]],
            rows_v.at[pl.ds(j * 128, 128)], sem))
        copies.append(pltpu.async_copy(
            fo_hbm.at[idx_v.at[pl.ds(j * 128, 128)]],
            fo_v.at[pl.ds(j * 128, 128)], sem))
      for cp in copies:
        cp.wait()

      def grp_body(g, gcarry):
        ids0 = iota * F + g * (L * F)
        zero = jnp.zeros((L,), jnp.float32)
        acc1 = zero
        acc_s = [zero] * D
        acc_q = [zero] * D
        for f in range(F):
          rid = ids0 + f
          fvv = plsc.load_gather(fv_v, [rid])
          wv = plsc.load_gather(fo_v, [rid])
          acc1 = acc1 + fvv * wv
          for d in range(D):
            e = plsc.load_gather(rows_v, [rid, jnp.full((L,), d, jnp.int32)])
            x = fvv * e
            acc_s[d] = acc_s[d] + x
            acc_q[d] = acc_q[d] + x * x
        s2 = jnp.zeros((L,), jnp.float32)
        sq = jnp.zeros((L,), jnp.float32)
        for d in range(D):
          s2 = s2 + acc_s[d] * acc_s[d]
          sq = sq + acc_q[d]
        t = bias_v[...] + acc1 + 0.5 * (s2 - sq)
        y = 1.0 / (1.0 + jnp.exp(-t))
        out_v[pl.ds(c * NB + g * L, L)] = y
        return gcarry

      lax.fori_loop(0, NGRP, grp_body, 0)
      return carry

    lax.fori_loop(0, NCHUNK, chunk_body, 0)
    pltpu.sync_copy(out_v, out_hbm.at[pl.ds(wid * RPW, RPW)])

  return fm


def kernel(feat_index, feat_value, emb_table, first_order_w, bias):
  B, F = feat_index.shape
  V, D = emb_table.shape
  fi = feat_index.astype(jnp.int32).reshape(B * F)
  fvf = feat_value.reshape(B * F).astype(jnp.float32)
  fo = first_order_w.reshape(V).astype(jnp.float32)
  bias16 = jnp.broadcast_to(bias.astype(jnp.float32), (16,))
  fm = _build_fm(B, F, V, D)
  return fm(fi, fvf, emb_table.astype(jnp.float32), fo, bias16)


# speedup vs baseline: 1.2848x; 1.1097x over previous
import functools

import jax
import jax.numpy as jnp
from jax import lax
from jax.experimental import pallas as pl
from jax.experimental.pallas import tpu as pltpu
from jax.experimental.pallas import tpu_sc as plsc


def _build_fm(B, F, V, D):
  info = plsc.get_sparse_core_info()
  NC, NS, L = info.num_cores, info.num_subcores, info.num_lanes
  NW = NC * NS
  assert D == L and B % (L * NW) == 0
  RPW = B // NW
  NB = 64
  NCHUNK = RPW // NB
  IPC = NB * F
  NGRP = NB // L

  mesh = plsc.VectorSubcoreMesh(core_axis_name="c", subcore_axis_name="s")

  @functools.partial(
      pl.kernel,
      mesh=mesh,
      compiler_params=pltpu.CompilerParams(
          needs_layout_passes=False, use_tc_tiling_on_sc=False),
      out_type=jax.ShapeDtypeStruct((B,), jnp.float32),
      scratch_types=[
          pltpu.VMEM((F, NB), jnp.int32),
          pltpu.VMEM((F, NB), jnp.float32),
          pltpu.VMEM((IPC,), jnp.float32),
          pltpu.VMEM((IPC, D), jnp.float32),
          pltpu.VMEM((L,), jnp.float32),
          pltpu.VMEM((RPW,), jnp.float32),
          pltpu.SemaphoreType.DMA,
      ],
  )
  def fm(fi_hbm, fv_hbm, emb_hbm, fo_hbm, bias_hbm, out_hbm,
         idx_v, fv_v, fo_v, rows_v, bias_v, out_v, sem):
    wid = lax.axis_index("s") * NC + lax.axis_index("c")
    pltpu.sync_copy(bias_hbm, bias_v)
    iota = lax.iota(jnp.int32, L)

    def chunk_body(c, carry):
      base = wid * RPW + c * NB
      pltpu.sync_copy(fi_hbm.at[:, pl.ds(base, NB)], idx_v)
      pltpu.sync_copy(fv_hbm.at[:, pl.ds(base, NB)], fv_v)
      copies = []
      for f in range(F):
        copies.append(pltpu.async_copy(
            emb_hbm.at[idx_v.at[f]], rows_v.at[pl.ds(f * NB, NB)], sem))
        copies.append(pltpu.async_copy(
            fo_hbm.at[idx_v.at[f]], fo_v.at[pl.ds(f * NB, NB)], sem))
      for cp in copies:
        cp.wait()

      def grp_body(g, gcarry):
        g16 = g * L
        zero = jnp.zeros((L,), jnp.float32)
        acc1 = zero
        acc_s = [zero] * D
        acc_q = [zero] * D
        for f in range(F):
          rid = iota + (f * NB + g16)
          fvv = fv_v[f, pl.ds(g16, L)]
          wv = fo_v[pl.ds(f * NB + g16, L)]
          acc1 = acc1 + fvv * wv
          for d in range(D):
            e = plsc.load_gather(rows_v, [rid, jnp.full((L,), d, jnp.int32)])
            x = fvv * e
            acc_s[d] = acc_s[d] + x
            acc_q[d] = acc_q[d] + x * x
        s2 = jnp.zeros((L,), jnp.float32)
        sq = jnp.zeros((L,), jnp.float32)
        for d in range(D):
          s2 = s2 + acc_s[d] * acc_s[d]
          sq = sq + acc_q[d]
        t = bias_v[...] + acc1 + 0.5 * (s2 - sq)
        y = 1.0 / (1.0 + jnp.exp(-t))
        out_v[pl.ds(c * NB + g16, L)] = y
        return gcarry

      lax.fori_loop(0, NGRP, grp_body, 0)
      return carry

    lax.fori_loop(0, NCHUNK, chunk_body, 0)
    pltpu.sync_copy(out_v, out_hbm.at[pl.ds(wid * RPW, RPW)])

  return fm


def kernel(feat_index, feat_value, emb_table, first_order_w, bias):
  B, F = feat_index.shape
  V, D = emb_table.shape
  fi_t = feat_index.astype(jnp.int32).T
  fv_t = feat_value.astype(jnp.float32).T
  fo = first_order_w.reshape(V).astype(jnp.float32)
  bias16 = jnp.broadcast_to(bias.astype(jnp.float32), (16,))
  fm = _build_fm(B, F, V, D)
  return fm(fi_t, fv_t, emb_table.astype(jnp.float32), fo, bias16)


# speedup vs baseline: 1.3507x; 1.0513x over previous
import functools

import jax
import jax.numpy as jnp
from jax import lax
from jax.experimental import pallas as pl
from jax.experimental.pallas import tpu as pltpu
from jax.experimental.pallas import tpu_sc as plsc


def _build_fm(B, F, V, D):
  info = plsc.get_sparse_core_info()
  NC, NS, L = info.num_cores, info.num_subcores, info.num_lanes
  NW = NC * NS
  assert D == L and B % (L * NW) == 0
  RPW = B // NW
  NB = 64
  NCHUNK = RPW // NB
  assert NCHUNK % 2 == 0
  IPC = NB * F
  NGRP = NB // L

  mesh = plsc.VectorSubcoreMesh(core_axis_name="c", subcore_axis_name="s")

  @functools.partial(
      pl.kernel,
      mesh=mesh,
      compiler_params=pltpu.CompilerParams(
          needs_layout_passes=False, use_tc_tiling_on_sc=False),
      out_type=jax.ShapeDtypeStruct((B,), jnp.float32),
      scratch_types=[
          pltpu.VMEM((F, NB), jnp.int32),
          pltpu.VMEM((F, NB), jnp.float32),
          pltpu.VMEM((IPC,), jnp.float32),
          pltpu.VMEM((IPC, D), jnp.float32),
          pltpu.VMEM((F, NB), jnp.int32),
          pltpu.VMEM((F, NB), jnp.float32),
          pltpu.VMEM((IPC,), jnp.float32),
          pltpu.VMEM((IPC, D), jnp.float32),
          pltpu.VMEM((L,), jnp.float32),
          pltpu.VMEM((RPW,), jnp.float32),
          pltpu.SemaphoreType.DMA,
          pltpu.SemaphoreType.DMA,
          pltpu.SemaphoreType.DMA,
          pltpu.SemaphoreType.DMA,
      ],
  )
  def fm(fi_hbm, fv_hbm, emb_hbm, fo_hbm, bias_hbm, out_hbm,
         idx_a, fv_a, fo_a, rows_a, idx_b, fv_b, fo_b, rows_b,
         bias_v, out_v, sem_ra, sem_fa, sem_rb, sem_fb):
    wid = lax.axis_index("s") * NC + lax.axis_index("c")
    pltpu.sync_copy(bias_hbm, bias_v)
    iota = lax.iota(jnp.int32, L)

    def fire(c, idx_v, fv_v, fo_v, rows_v, sem_r, sem_f):
      base = wid * RPW + c * NB
      pltpu.sync_copy(fi_hbm.at[:, pl.ds(base, NB)], idx_v)
      pltpu.sync_copy(fv_hbm.at[:, pl.ds(base, NB)], fv_v)
      for f in range(F):
        pltpu.async_copy(
            emb_hbm.at[idx_v.at[f]], rows_v.at[pl.ds(f * NB, NB)], sem_r)
        pltpu.async_copy(
            fo_hbm.at[idx_v.at[f]], fo_v.at[pl.ds(f * NB, NB)], sem_f)

    def drain(idx_v, fo_v, rows_v, sem_r, sem_f):
      for f in range(F):
        pltpu.make_async_copy(
            emb_hbm.at[idx_v.at[f]], rows_v.at[pl.ds(f * NB, NB)],
            sem_r).wait()
        pltpu.make_async_copy(
            fo_hbm.at[idx_v.at[f]], fo_v.at[pl.ds(f * NB, NB)],
            sem_f).wait()

    def compute(c, fv_v, fo_v, rows_v):
      def grp_body(g, gcarry):
        g16 = g * L
        zero = jnp.zeros((L,), jnp.float32)
        acc1 = zero
        acc_s = [zero] * D
        acc_q = [zero] * D
        for f in range(F):
          rid = iota + (f * NB + g16)
          fvv = fv_v[f, pl.ds(g16, L)]
          wv = fo_v[pl.ds(f * NB + g16, L)]
          acc1 = acc1 + fvv * wv
          for d in range(D):
            e = plsc.load_gather(rows_v, [rid, jnp.full((L,), d, jnp.int32)])
            x = fvv * e
            acc_s[d] = acc_s[d] + x
            acc_q[d] = acc_q[d] + x * x
        s2 = jnp.zeros((L,), jnp.float32)
        sq = jnp.zeros((L,), jnp.float32)
        for d in range(D):
          s2 = s2 + acc_s[d] * acc_s[d]
          sq = sq + acc_q[d]
        t = bias_v[...] + acc1 + 0.5 * (s2 - sq)
        y = 1.0 / (1.0 + jnp.exp(-t))
        out_v[pl.ds(c * NB + g16, L)] = y
        return gcarry

      lax.fori_loop(0, NGRP, grp_body, 0)

    fire(0, idx_a, fv_a, fo_a, rows_a, sem_ra, sem_fa)

    def pair_body(k, carry):
      c0 = 2 * k
      fire(c0 + 1, idx_b, fv_b, fo_b, rows_b, sem_rb, sem_fb)
      drain(idx_a, fo_a, rows_a, sem_ra, sem_fa)
      compute(c0, fv_a, fo_a, rows_a)

      @pl.when(k < NCHUNK // 2 - 1)
      def _():
        fire(c0 + 2, idx_a, fv_a, fo_a, rows_a, sem_ra, sem_fa)

      drain(idx_b, fo_b, rows_b, sem_rb, sem_fb)
      compute(c0 + 1, fv_b, fo_b, rows_b)
      return carry

    lax.fori_loop(0, NCHUNK // 2, pair_body, 0)
    pltpu.sync_copy(out_v, out_hbm.at[pl.ds(wid * RPW, RPW)])

  return fm


def kernel(feat_index, feat_value, emb_table, first_order_w, bias):
  B, F = feat_index.shape
  V, D = emb_table.shape
  fi_t = feat_index.astype(jnp.int32).T
  fv_t = feat_value.astype(jnp.float32).T
  fo = first_order_w.reshape(V).astype(jnp.float32)
  bias16 = jnp.broadcast_to(bias.astype(jnp.float32), (16,))
  fm = _build_fm(B, F, V, D)
  return fm(fi_t, fv_t, emb_table.astype(jnp.float32), fo, bias16)


# speedup vs baseline: 2.9945x; 2.2170x over previous
import functools

import jax
import jax.numpy as jnp
from jax import lax
from jax.experimental import pallas as pl
from jax.experimental.pallas import tpu as pltpu
from jax.experimental.pallas import tpu_sc as plsc


def _build_relayout(V, D):
  info = plsc.get_sparse_core_info()
  NC, NS, L = info.num_cores, info.num_subcores, info.num_lanes
  NW = NC * NS
  assert D == L
  NBLK = -(-V // 128)
  NFULL = V // 128
  TAILV = V % 128
  S = NBLK - NW
  NMAIN = -(-S // NW)
  assert NMAIN % 2 == 0 and NMAIN * NW <= NFULL and TAILV % 16 == 0
  BW = 128 * D

  mesh = plsc.VectorSubcoreMesh(core_axis_name="c", subcore_axis_name="s")

  @functools.partial(
      pl.kernel,
      mesh=mesh,
      compiler_params=pltpu.CompilerParams(
          needs_layout_passes=False, use_tc_tiling_on_sc=True),
      out_type=jax.ShapeDtypeStruct((V * D,), jnp.float32),
      scratch_types=[
          pltpu.VMEM((D, 128), jnp.float32),
          pltpu.VMEM((D, 128), jnp.float32),
          pltpu.VMEM((BW,), jnp.float32),
          pltpu.VMEM((BW,), jnp.float32),
          pltpu.SemaphoreType.DMA,
          pltpu.SemaphoreType.DMA,
          pltpu.SemaphoreType.DMA,
          pltpu.SemaphoreType.DMA,
      ],
  )
  def relayout(src_hbm, out_hbm, st0, st1, ov0, ov1, si0, si1, so0, so1):
    wid = lax.axis_index("s") * NC + lax.axis_index("c")
    iota16 = lax.iota(jnp.int32, L) * D
    stages = (st0, st1)
    outvs = (ov0, ov1)
    sin = (si0, si1)
    sout = (so0, so1)

    def fire_in(b, p):
      pltpu.async_copy(
          src_hbm.at[:, pl.ds(b * 128, 128)], stages[p], sin[p])

    def transpose(p):
      for g in range(8):
        xs = [stages[p][d, pl.ds(g * L, L)] for d in range(D)]
        for d in range(D):
          plsc.store_scatter(outvs[p], [iota16 + (g * 256 + d)], xs[d])

    fire_in(wid, 0)
    fire_in(NW + wid, 1)

    def body(k, carry):
      for p in range(2):
        i = 2 * k + p
        b = i * NW + wid
        pltpu.make_async_copy(
            src_hbm.at[:, pl.ds(b * 128, 128)], stages[p], sin[p]).wait()

        @pl.when(k > 0)
        def _():
          pltpu.make_async_copy(
              outvs[p], out_hbm.at[pl.ds(b * BW, BW)], sout[p]).wait()

        transpose(p)
        pltpu.async_copy(outvs[p], out_hbm.at[pl.ds(b * BW, BW)], sout[p])

        @pl.when(i + 2 <= NMAIN - 1)
        def _():
          fire_in((i + 2) * NW + wid, p)
      return carry

    lax.fori_loop(0, NMAIN // 2, body, 0)
    for p in range(2):
      pltpu.make_async_copy(
          outvs[p], out_hbm.at[pl.ds(0, BW)], sout[p]).wait()

    b = S + wid
    pltpu.sync_copy(src_hbm.at[:, pl.ds(b * 128, 128)], st0)
    transpose(0)

    @pl.when(b < NFULL)
    def _():
      pltpu.sync_copy(ov0, out_hbm.at[pl.ds(b * BW, BW)])

    @pl.when(b == NFULL)
    def _():
      pltpu.sync_copy(
          ov0.at[pl.ds(0, TAILV * D)], out_hbm.at[pl.ds(b * BW, TAILV * D)])

  return relayout


def _build_fm(B, F, V, D):
  info = plsc.get_sparse_core_info()
  NC, NS, L = info.num_cores, info.num_subcores, info.num_lanes
  NW = NC * NS
  assert D == L and B % (L * NW) == 0
  RPW = B // NW
  NB = 64
  NCHUNK = RPW // NB
  assert NCHUNK % 2 == 0
  IPC = NB * F
  NGRP = NB // L

  mesh = plsc.VectorSubcoreMesh(core_axis_name="c", subcore_axis_name="s")

  @functools.partial(
      pl.kernel,
      mesh=mesh,
      compiler_params=pltpu.CompilerParams(
          needs_layout_passes=False, use_tc_tiling_on_sc=False),
      out_type=jax.ShapeDtypeStruct((B,), jnp.float32),
      scratch_types=[
          pltpu.VMEM((F, NB), jnp.int32),
          pltpu.VMEM((F, NB), jnp.float32),
          pltpu.VMEM((IPC,), jnp.float32),
          pltpu.VMEM((IPC, D), jnp.float32),
          pltpu.VMEM((F, NB), jnp.int32),
          pltpu.VMEM((F, NB), jnp.float32),
          pltpu.VMEM((IPC,), jnp.float32),
          pltpu.VMEM((IPC, D), jnp.float32),
          pltpu.VMEM((L,), jnp.float32),
          pltpu.VMEM((RPW,), jnp.float32),
          pltpu.SemaphoreType.DMA,
          pltpu.SemaphoreType.DMA,
          pltpu.SemaphoreType.DMA,
          pltpu.SemaphoreType.DMA,
      ],
  )
  def fm(fi_hbm, fv_hbm, emb_hbm, fo_hbm, bias_hbm, out_hbm,
         idx_a, fv_a, fo_a, rows_a, idx_b, fv_b, fo_b, rows_b,
         bias_v, out_v, sem_ra, sem_fa, sem_rb, sem_fb):
    wid = lax.axis_index("s") * NC + lax.axis_index("c")
    pltpu.sync_copy(bias_hbm, bias_v)
    iota = lax.iota(jnp.int32, L)

    def fire(c, idx_v, fv_v, fo_v, rows_v, sem_r, sem_f):
      base = wid * RPW + c * NB
      pltpu.sync_copy(fi_hbm.at[:, pl.ds(base, NB)], idx_v)
      pltpu.sync_copy(fv_hbm.at[:, pl.ds(base, NB)], fv_v)
      for f in range(F):
        pltpu.async_copy(
            emb_hbm.at[idx_v.at[f]], rows_v.at[pl.ds(f * NB, NB)], sem_r)
        pltpu.async_copy(
            fo_hbm.at[idx_v.at[f]], fo_v.at[pl.ds(f * NB, NB)], sem_f)

    def drain(idx_v, fo_v, rows_v, sem_r, sem_f):
      for f in range(F):
        pltpu.make_async_copy(
            emb_hbm.at[idx_v.at[f]], rows_v.at[pl.ds(f * NB, NB)],
            sem_r).wait()
        pltpu.make_async_copy(
            fo_hbm.at[idx_v.at[f]], fo_v.at[pl.ds(f * NB, NB)],
            sem_f).wait()

    def compute(c, fv_v, fo_v, rows_v):
      def grp_body(g, gcarry):
        g16 = g * L
        zero = jnp.zeros((L,), jnp.float32)
        acc1 = zero
        acc_s = [zero] * D
        acc_q = [zero] * D
        for f in range(F):
          rid = iota + (f * NB + g16)
          fvv = fv_v[f, pl.ds(g16, L)]
          wv = fo_v[pl.ds(f * NB + g16, L)]
          acc1 = acc1 + fvv * wv
          for d in range(D):
            e = plsc.load_gather(rows_v, [rid, jnp.full((L,), d, jnp.int32)])
            x = fvv * e
            acc_s[d] = acc_s[d] + x
            acc_q[d] = acc_q[d] + x * x
        s2 = jnp.zeros((L,), jnp.float32)
        sq = jnp.zeros((L,), jnp.float32)
        for d in range(D):
          s2 = s2 + acc_s[d] * acc_s[d]
          sq = sq + acc_q[d]
        t = bias_v[...] + acc1 + 0.5 * (s2 - sq)
        y = 1.0 / (1.0 + jnp.exp(-t))
        out_v[pl.ds(c * NB + g16, L)] = y
        return gcarry

      lax.fori_loop(0, NGRP, grp_body, 0)

    fire(0, idx_a, fv_a, fo_a, rows_a, sem_ra, sem_fa)

    def pair_body(k, carry):
      c0 = 2 * k
      fire(c0 + 1, idx_b, fv_b, fo_b, rows_b, sem_rb, sem_fb)
      drain(idx_a, fo_a, rows_a, sem_ra, sem_fa)
      compute(c0, fv_a, fo_a, rows_a)

      @pl.when(k < NCHUNK // 2 - 1)
      def _():
        fire(c0 + 2, idx_a, fv_a, fo_a, rows_a, sem_ra, sem_fa)

      drain(idx_b, fo_b, rows_b, sem_rb, sem_fb)
      compute(c0 + 1, fv_b, fo_b, rows_b)
      return carry

    lax.fori_loop(0, NCHUNK // 2, pair_body, 0)
    pltpu.sync_copy(out_v, out_hbm.at[pl.ds(wid * RPW, RPW)])

  return fm


def kernel(feat_index, feat_value, emb_table, first_order_w, bias):
  B, F = feat_index.shape
  V, D = emb_table.shape
  fi_t = feat_index.astype(jnp.int32).T
  fv_t = feat_value.astype(jnp.float32).T
  emb_t = emb_table.astype(jnp.float32).T
  fo = first_order_w.astype(jnp.float32).T.reshape(V)
  bias16 = jnp.broadcast_to(bias.astype(jnp.float32), (16,))
  emb_lin = _build_relayout(V, D)(emb_t).reshape(V, D)
  fm = _build_fm(B, F, V, D)
  return fm(fi_t, fv_t, emb_lin, fo, bias16)


# speedup vs baseline: 3.0323x; 1.0126x over previous
import functools

import jax
import jax.numpy as jnp
from jax import lax
from jax.experimental import pallas as pl
from jax.experimental.pallas import tpu as pltpu
from jax.experimental.pallas import tpu_sc as plsc


def _build_relayout(V, D):
  info = plsc.get_sparse_core_info()
  NC, NS, L = info.num_cores, info.num_subcores, info.num_lanes
  NW = NC * NS
  assert D == L
  NBLK = -(-V // 128)
  NFULL = V // 128
  TAILV = V % 128
  S = NBLK - NW
  NMAIN = -(-S // NW)
  assert NMAIN % 2 == 0 and NMAIN * NW <= NFULL and TAILV % 16 == 0
  BW = 128 * D

  mesh = plsc.VectorSubcoreMesh(core_axis_name="c", subcore_axis_name="s")

  @functools.partial(
      pl.kernel,
      mesh=mesh,
      compiler_params=pltpu.CompilerParams(
          needs_layout_passes=False, use_tc_tiling_on_sc=True),
      out_type=jax.ShapeDtypeStruct((V * D,), jnp.float32),
      scratch_types=[
          pltpu.VMEM((D, 128), jnp.float32),
          pltpu.VMEM((D, 128), jnp.float32),
          pltpu.VMEM((BW,), jnp.float32),
          pltpu.VMEM((BW,), jnp.float32),
          pltpu.SemaphoreType.DMA,
          pltpu.SemaphoreType.DMA,
          pltpu.SemaphoreType.DMA,
          pltpu.SemaphoreType.DMA,
      ],
  )
  def relayout(src_hbm, out_hbm, st0, st1, ov0, ov1, si0, si1, so0, so1):
    wid = lax.axis_index("s") * NC + lax.axis_index("c")
    iota16 = lax.iota(jnp.int32, L) * D
    stages = (st0, st1)
    outvs = (ov0, ov1)
    sin = (si0, si1)
    sout = (so0, so1)

    def fire_in(b, p):
      pltpu.async_copy(
          src_hbm.at[:, pl.ds(b * 128, 128)], stages[p], sin[p])

    def transpose(p):
      st, ov = stages[p], outvs[p]
      xs = [st[d, pl.ds(0, L)] for d in range(D)]
      for g in range(8):
        nxt = [st[d, pl.ds((g + 1) * L, L)] for d in range(D)] if g < 7 else []
        for d in range(D):
          plsc.store_scatter(ov, [iota16 + (g * 256 + d)], xs[d])
        xs = nxt

    fire_in(wid, 0)
    fire_in(NW + wid, 1)

    def body(k, carry):
      for p in range(2):
        i = 2 * k + p
        b = i * NW + wid
        pltpu.make_async_copy(
            src_hbm.at[:, pl.ds(b * 128, 128)], stages[p], sin[p]).wait()

        @pl.when(k > 0)
        def _():
          pltpu.make_async_copy(
              outvs[p], out_hbm.at[pl.ds(b * BW, BW)], sout[p]).wait()

        transpose(p)
        pltpu.async_copy(outvs[p], out_hbm.at[pl.ds(b * BW, BW)], sout[p])

        @pl.when(i + 2 <= NMAIN - 1)
        def _():
          fire_in((i + 2) * NW + wid, p)
      return carry

    lax.fori_loop(0, NMAIN // 2, body, 0)
    for p in range(2):
      pltpu.make_async_copy(
          outvs[p], out_hbm.at[pl.ds(0, BW)], sout[p]).wait()

    b = S + wid
    pltpu.sync_copy(src_hbm.at[:, pl.ds(b * 128, 128)], st0)
    transpose(0)

    @pl.when(b < NFULL)
    def _():
      pltpu.sync_copy(ov0, out_hbm.at[pl.ds(b * BW, BW)])

    @pl.when(b == NFULL)
    def _():
      pltpu.sync_copy(
          ov0.at[pl.ds(0, TAILV * D)], out_hbm.at[pl.ds(b * BW, TAILV * D)])

  return relayout


def _build_fm(B, F, V, D):
  info = plsc.get_sparse_core_info()
  NC, NS, L = info.num_cores, info.num_subcores, info.num_lanes
  NW = NC * NS
  assert D == L and B % (L * NW) == 0
  RPW = B // NW
  NB = 64
  NCHUNK = RPW // NB
  assert NCHUNK % 2 == 0
  IPC = NB * F
  NGRP = NB // L

  mesh = plsc.VectorSubcoreMesh(core_axis_name="c", subcore_axis_name="s")

  @functools.partial(
      pl.kernel,
      mesh=mesh,
      compiler_params=pltpu.CompilerParams(
          needs_layout_passes=False, use_tc_tiling_on_sc=False),
      out_type=jax.ShapeDtypeStruct((B,), jnp.float32),
      scratch_types=[
          pltpu.VMEM((F, NB), jnp.int32),
          pltpu.VMEM((F, NB), jnp.float32),
          pltpu.VMEM((IPC,), jnp.float32),
          pltpu.VMEM((IPC, D), jnp.float32),
          pltpu.VMEM((F, NB), jnp.int32),
          pltpu.VMEM((F, NB), jnp.float32),
          pltpu.VMEM((IPC,), jnp.float32),
          pltpu.VMEM((IPC, D), jnp.float32),
          pltpu.VMEM((L,), jnp.float32),
          pltpu.VMEM((RPW,), jnp.float32),
          pltpu.SemaphoreType.DMA,
          pltpu.SemaphoreType.DMA,
          pltpu.SemaphoreType.DMA,
          pltpu.SemaphoreType.DMA,
      ],
  )
  def fm(fi_hbm, fv_hbm, emb_hbm, fo_hbm, bias_hbm, out_hbm,
         idx_a, fv_a, fo_a, rows_a, idx_b, fv_b, fo_b, rows_b,
         bias_v, out_v, sem_ra, sem_fa, sem_rb, sem_fb):
    wid = lax.axis_index("s") * NC + lax.axis_index("c")
    pltpu.sync_copy(bias_hbm, bias_v)
    iota = lax.iota(jnp.int32, L)

    def fire(c, idx_v, fv_v, fo_v, rows_v, sem_r, sem_f):
      base = wid * RPW + c * NB
      pltpu.sync_copy(fi_hbm.at[:, pl.ds(base, NB)], idx_v)
      pltpu.sync_copy(fv_hbm.at[:, pl.ds(base, NB)], fv_v)
      for f in range(F):
        pltpu.async_copy(
            emb_hbm.at[idx_v.at[f]], rows_v.at[pl.ds(f * NB, NB)], sem_r)
        pltpu.async_copy(
            fo_hbm.at[idx_v.at[f]], fo_v.at[pl.ds(f * NB, NB)], sem_f)

    def drain(idx_v, fo_v, rows_v, sem_r, sem_f):
      for f in range(F):
        pltpu.make_async_copy(
            emb_hbm.at[idx_v.at[f]], rows_v.at[pl.ds(f * NB, NB)],
            sem_r).wait()
        pltpu.make_async_copy(
            fo_hbm.at[idx_v.at[f]], fo_v.at[pl.ds(f * NB, NB)],
            sem_f).wait()

    def compute(c, fv_v, fo_v, rows_v):
      def grp_body(g, gcarry):
        g16 = g * L
        zero = jnp.zeros((L,), jnp.float32)
        acc1 = zero
        acc_s = [zero] * D
        acc_q = [zero] * D
        for f in range(F):
          rid = iota + (f * NB + g16)
          fvv = fv_v[f, pl.ds(g16, L)]
          wv = fo_v[pl.ds(f * NB + g16, L)]
          acc1 = acc1 + fvv * wv
          for d in range(D):
            e = plsc.load_gather(rows_v, [rid, jnp.full((L,), d, jnp.int32)])
            x = fvv * e
            acc_s[d] = acc_s[d] + x
            acc_q[d] = acc_q[d] + x * x
        s2 = jnp.zeros((L,), jnp.float32)
        sq = jnp.zeros((L,), jnp.float32)
        for d in range(D):
          s2 = s2 + acc_s[d] * acc_s[d]
          sq = sq + acc_q[d]
        t = bias_v[...] + acc1 + 0.5 * (s2 - sq)
        y = 1.0 / (1.0 + jnp.exp(-t))
        out_v[pl.ds(c * NB + g16, L)] = y
        return gcarry

      lax.fori_loop(0, NGRP, grp_body, 0)

    fire(0, idx_a, fv_a, fo_a, rows_a, sem_ra, sem_fa)

    def pair_body(k, carry):
      c0 = 2 * k
      fire(c0 + 1, idx_b, fv_b, fo_b, rows_b, sem_rb, sem_fb)
      drain(idx_a, fo_a, rows_a, sem_ra, sem_fa)
      compute(c0, fv_a, fo_a, rows_a)

      @pl.when(k < NCHUNK // 2 - 1)
      def _():
        fire(c0 + 2, idx_a, fv_a, fo_a, rows_a, sem_ra, sem_fa)

      drain(idx_b, fo_b, rows_b, sem_rb, sem_fb)
      compute(c0 + 1, fv_b, fo_b, rows_b)
      return carry

    lax.fori_loop(0, NCHUNK // 2, pair_body, 0)
    pltpu.sync_copy(out_v, out_hbm.at[pl.ds(wid * RPW, RPW)])

  return fm


def kernel(feat_index, feat_value, emb_table, first_order_w, bias):
  B, F = feat_index.shape
  V, D = emb_table.shape
  fi_t = feat_index.astype(jnp.int32).T
  fv_t = feat_value.astype(jnp.float32).T
  emb_t = emb_table.astype(jnp.float32).T
  fo = first_order_w.astype(jnp.float32).T.reshape(V)
  bias16 = jnp.broadcast_to(bias.astype(jnp.float32), (16,))
  emb_lin = _build_relayout(V, D)(emb_t).reshape(V, D)
  fm = _build_fm(B, F, V, D)
  return fm(fi_t, fv_t, emb_lin, fo, bias16)


# speedup vs baseline: 3.0617x; 1.0097x over previous
import functools

import jax
import jax.numpy as jnp
from jax import lax
from jax.experimental import pallas as pl
from jax.experimental.pallas import tpu as pltpu
from jax.experimental.pallas import tpu_sc as plsc


def _build_relayout(V, D):
  info = plsc.get_sparse_core_info()
  NC, NS, L = info.num_cores, info.num_subcores, info.num_lanes
  NW = NC * NS
  assert D == L
  NBLK = -(-V // 128)
  NFULL = V // 128
  TAILV = V % 128
  S = NBLK - NW
  NMAIN = -(-S // NW)
  assert NMAIN % 2 == 0 and NMAIN * NW <= NFULL and TAILV % 16 == 0
  BW = 128 * D

  mesh = plsc.VectorSubcoreMesh(core_axis_name="c", subcore_axis_name="s")

  @functools.partial(
      pl.kernel,
      mesh=mesh,
      compiler_params=pltpu.CompilerParams(
          needs_layout_passes=False, use_tc_tiling_on_sc=True),
      out_type=jax.ShapeDtypeStruct((V * D,), jnp.float32),
      scratch_types=[
          pltpu.VMEM((D, 128), jnp.float32),
          pltpu.VMEM((D, 128), jnp.float32),
          pltpu.VMEM((BW,), jnp.float32),
          pltpu.VMEM((BW,), jnp.float32),
          pltpu.SemaphoreType.DMA,
          pltpu.SemaphoreType.DMA,
          pltpu.SemaphoreType.DMA,
          pltpu.SemaphoreType.DMA,
      ],
  )
  def relayout(src_hbm, out_hbm, st0, st1, ov0, ov1, si0, si1, so0, so1):
    wid = lax.axis_index("s") * NC + lax.axis_index("c")
    iota16 = lax.iota(jnp.int32, L) * D
    stages = (st0, st1)
    outvs = (ov0, ov1)
    sin = (si0, si1)
    sout = (so0, so1)

    def fire_in(b, p):
      pltpu.async_copy(
          src_hbm.at[:, pl.ds(b * 128, 128)], stages[p], sin[p])

    def transpose(p):
      st, ov = stages[p], outvs[p]
      xs = [st[d, pl.ds(0, L)] for d in range(D)]
      for g in range(8):
        nxt = [st[d, pl.ds((g + 1) * L, L)] for d in range(D)] if g < 7 else []
        for d in range(D):
          plsc.store_scatter(ov, [iota16 + (g * 256 + d)], xs[d])
        xs = nxt

    fire_in(wid, 0)
    fire_in(NW + wid, 1)

    def body(k, carry):
      for p in range(2):
        i = 2 * k + p
        b = i * NW + wid
        pltpu.make_async_copy(
            src_hbm.at[:, pl.ds(b * 128, 128)], stages[p], sin[p]).wait()

        @pl.when(k > 0)
        def _():
          pltpu.make_async_copy(
              outvs[p], out_hbm.at[pl.ds(b * BW, BW)], sout[p]).wait()

        transpose(p)
        pltpu.async_copy(outvs[p], out_hbm.at[pl.ds(b * BW, BW)], sout[p])

        @pl.when(i + 2 <= NMAIN - 1)
        def _():
          fire_in((i + 2) * NW + wid, p)
      return carry

    lax.fori_loop(0, NMAIN // 2, body, 0)
    for p in range(2):
      pltpu.make_async_copy(
          outvs[p], out_hbm.at[pl.ds(0, BW)], sout[p]).wait()

    b = S + wid
    pltpu.sync_copy(src_hbm.at[:, pl.ds(b * 128, 128)], st0)
    transpose(0)

    @pl.when(b < NFULL)
    def _():
      pltpu.sync_copy(ov0, out_hbm.at[pl.ds(b * BW, BW)])

    @pl.when(b == NFULL)
    def _():
      pltpu.sync_copy(
          ov0.at[pl.ds(0, TAILV * D)], out_hbm.at[pl.ds(b * BW, TAILV * D)])

  return relayout


def _build_fm(B, F, V, D):
  info = plsc.get_sparse_core_info()
  NC, NS, L = info.num_cores, info.num_subcores, info.num_lanes
  NW = NC * NS
  assert D == L and B % (L * NW) == 0
  RPW = B // NW
  NB = 128
  NCHUNK = RPW // NB
  assert NCHUNK % 2 == 0
  IPC = NB * F
  NGRP = NB // L

  mesh = plsc.VectorSubcoreMesh(core_axis_name="c", subcore_axis_name="s")

  @functools.partial(
      pl.kernel,
      mesh=mesh,
      compiler_params=pltpu.CompilerParams(
          needs_layout_passes=False, use_tc_tiling_on_sc=False),
      out_type=jax.ShapeDtypeStruct((B,), jnp.float32),
      scratch_types=[
          pltpu.VMEM((F, NB), jnp.int32),
          pltpu.VMEM((F, NB), jnp.float32),
          pltpu.VMEM((IPC,), jnp.float32),
          pltpu.VMEM((IPC, D), jnp.float32),
          pltpu.VMEM((F, NB), jnp.int32),
          pltpu.VMEM((F, NB), jnp.float32),
          pltpu.VMEM((IPC,), jnp.float32),
          pltpu.VMEM((IPC, D), jnp.float32),
          pltpu.VMEM((L,), jnp.float32),
          pltpu.VMEM((RPW,), jnp.float32),
          pltpu.SemaphoreType.DMA,
          pltpu.SemaphoreType.DMA,
          pltpu.SemaphoreType.DMA,
          pltpu.SemaphoreType.DMA,
      ],
  )
  def fm(fi_hbm, fv_hbm, emb_hbm, fo_hbm, bias_hbm, out_hbm,
         idx_a, fv_a, fo_a, rows_a, idx_b, fv_b, fo_b, rows_b,
         bias_v, out_v, sem_ra, sem_fa, sem_rb, sem_fb):
    wid = lax.axis_index("s") * NC + lax.axis_index("c")
    pltpu.sync_copy(bias_hbm, bias_v)
    iota = lax.iota(jnp.int32, L)

    def fire(c, idx_v, fv_v, fo_v, rows_v, sem_r, sem_f):
      base = wid * RPW + c * NB
      pltpu.sync_copy(fi_hbm.at[:, pl.ds(base, NB)], idx_v)
      pltpu.sync_copy(fv_hbm.at[:, pl.ds(base, NB)], fv_v)
      for f in range(F):
        pltpu.async_copy(
            emb_hbm.at[idx_v.at[f]], rows_v.at[pl.ds(f * NB, NB)], sem_r)
        pltpu.async_copy(
            fo_hbm.at[idx_v.at[f]], fo_v.at[pl.ds(f * NB, NB)], sem_f)

    def drain(idx_v, fo_v, rows_v, sem_r, sem_f):
      for f in range(F):
        pltpu.make_async_copy(
            emb_hbm.at[idx_v.at[f]], rows_v.at[pl.ds(f * NB, NB)],
            sem_r).wait()
        pltpu.make_async_copy(
            fo_hbm.at[idx_v.at[f]], fo_v.at[pl.ds(f * NB, NB)],
            sem_f).wait()

    def compute(c, fv_v, fo_v, rows_v):
      def grp_body(g, gcarry):
        g16 = g * L
        zero = jnp.zeros((L,), jnp.float32)
        acc1 = zero
        acc_s = [zero] * D
        acc_q = [zero] * D
        for f in range(F):
          rid = iota + (f * NB + g16)
          fvv = fv_v[f, pl.ds(g16, L)]
          wv = fo_v[pl.ds(f * NB + g16, L)]
          acc1 = acc1 + fvv * wv
          for d in range(D):
            e = plsc.load_gather(rows_v, [rid, jnp.full((L,), d, jnp.int32)])
            x = fvv * e
            acc_s[d] = acc_s[d] + x
            acc_q[d] = acc_q[d] + x * x
        s2 = jnp.zeros((L,), jnp.float32)
        sq = jnp.zeros((L,), jnp.float32)
        for d in range(D):
          s2 = s2 + acc_s[d] * acc_s[d]
          sq = sq + acc_q[d]
        t = bias_v[...] + acc1 + 0.5 * (s2 - sq)
        y = 1.0 / (1.0 + jnp.exp(-t))
        out_v[pl.ds(c * NB + g16, L)] = y
        return gcarry

      lax.fori_loop(0, NGRP, grp_body, 0)

    fire(0, idx_a, fv_a, fo_a, rows_a, sem_ra, sem_fa)

    def pair_body(k, carry):
      c0 = 2 * k
      fire(c0 + 1, idx_b, fv_b, fo_b, rows_b, sem_rb, sem_fb)
      drain(idx_a, fo_a, rows_a, sem_ra, sem_fa)
      compute(c0, fv_a, fo_a, rows_a)

      @pl.when(k < NCHUNK // 2 - 1)
      def _():
        fire(c0 + 2, idx_a, fv_a, fo_a, rows_a, sem_ra, sem_fa)

      drain(idx_b, fo_b, rows_b, sem_rb, sem_fb)
      compute(c0 + 1, fv_b, fo_b, rows_b)
      return carry

    lax.fori_loop(0, NCHUNK // 2, pair_body, 0)
    pltpu.sync_copy(out_v, out_hbm.at[pl.ds(wid * RPW, RPW)])

  return fm


def kernel(feat_index, feat_value, emb_table, first_order_w, bias):
  B, F = feat_index.shape
  V, D = emb_table.shape
  fi_t = feat_index.astype(jnp.int32).T
  fv_t = feat_value.astype(jnp.float32).T
  emb_t = emb_table.astype(jnp.float32).T
  fo = first_order_w.astype(jnp.float32).T.reshape(V)
  bias16 = jnp.broadcast_to(bias.astype(jnp.float32), (16,))
  emb_lin = _build_relayout(V, D)(emb_t).reshape(V, D)
  fm = _build_fm(B, F, V, D)
  return fm(fi_t, fv_t, emb_lin, fo, bias16)


# speedup vs baseline: 3.1576x; 1.0313x over previous
import functools

import jax
import jax.numpy as jnp
from jax import lax
from jax.experimental import pallas as pl
from jax.experimental.pallas import tpu as pltpu
from jax.experimental.pallas import tpu_sc as plsc


def _build_relayout(V, D):
  info = plsc.get_sparse_core_info()
  NC, NS, L = info.num_cores, info.num_subcores, info.num_lanes
  NW = NC * NS
  assert D == L
  NBLK = -(-V // 128)
  NFULL = V // 128
  TAILV = V % 128
  S = NBLK - NW
  NMAIN = -(-S // NW)
  assert NMAIN % 2 == 0 and NMAIN * NW <= NFULL and TAILV % 16 == 0
  BW = 128 * D

  mesh = plsc.VectorSubcoreMesh(core_axis_name="c", subcore_axis_name="s")

  @functools.partial(
      pl.kernel,
      mesh=mesh,
      compiler_params=pltpu.CompilerParams(
          needs_layout_passes=False, use_tc_tiling_on_sc=True),
      out_type=jax.ShapeDtypeStruct((V * D,), jnp.float32),
      scratch_types=[
          pltpu.VMEM((D, 128), jnp.float32),
          pltpu.VMEM((D, 128), jnp.float32),
          pltpu.VMEM((BW,), jnp.float32),
          pltpu.VMEM((BW,), jnp.float32),
          pltpu.SemaphoreType.DMA,
          pltpu.SemaphoreType.DMA,
          pltpu.SemaphoreType.DMA,
          pltpu.SemaphoreType.DMA,
      ],
  )
  def relayout(src_hbm, out_hbm, st0, st1, ov0, ov1, si0, si1, so0, so1):
    wid = lax.axis_index("s") * NC + lax.axis_index("c")
    iota16 = lax.iota(jnp.int32, L) * D
    stages = (st0, st1)
    outvs = (ov0, ov1)
    sin = (si0, si1)
    sout = (so0, so1)

    def fire_in(b, p):
      pltpu.async_copy(
          src_hbm.at[:, pl.ds(b * 128, 128)], stages[p], sin[p])

    def transpose(p):
      st, ov = stages[p], outvs[p]
      for g in range(8):
        xs = [st[d, pl.ds(g * L, L)] for d in range(D)]
        for d in range(D):
          plsc.store_scatter(ov, [iota16 + (g * 256 + d)], xs[d])

    fire_in(wid, 0)
    fire_in(NW + wid, 1)

    def body(k, carry):
      for p in range(2):
        i = 2 * k + p
        b = i * NW + wid
        pltpu.make_async_copy(
            src_hbm.at[:, pl.ds(b * 128, 128)], stages[p], sin[p]).wait()

        @pl.when(k > 0)
        def _():
          pltpu.make_async_copy(
              outvs[p], out_hbm.at[pl.ds(b * BW, BW)], sout[p]).wait()

        transpose(p)
        pltpu.async_copy(outvs[p], out_hbm.at[pl.ds(b * BW, BW)], sout[p])

        @pl.when(i + 2 <= NMAIN - 1)
        def _():
          fire_in((i + 2) * NW + wid, p)
      return carry

    lax.fori_loop(0, NMAIN // 2, body, 0)
    for p in range(2):
      pltpu.make_async_copy(
          outvs[p], out_hbm.at[pl.ds(0, BW)], sout[p]).wait()

    b = S + wid
    pltpu.sync_copy(src_hbm.at[:, pl.ds(b * 128, 128)], st0)
    transpose(0)

    @pl.when(b < NFULL)
    def _():
      pltpu.sync_copy(ov0, out_hbm.at[pl.ds(b * BW, BW)])

    @pl.when(b == NFULL)
    def _():
      pltpu.sync_copy(
          ov0.at[pl.ds(0, TAILV * D)], out_hbm.at[pl.ds(b * BW, TAILV * D)])

  return relayout


def _build_fm(B, F, V, D):
  info = plsc.get_sparse_core_info()
  NC, NS, L = info.num_cores, info.num_subcores, info.num_lanes
  NW = NC * NS
  assert D == L and B % (L * NW) == 0
  RPW = B // NW
  NB = 128
  NCHUNK = RPW // NB
  assert NCHUNK % 2 == 0
  IPC = NB * F
  NGRP = NB // L

  mesh = plsc.VectorSubcoreMesh(core_axis_name="c", subcore_axis_name="s")

  @functools.partial(
      pl.kernel,
      mesh=mesh,
      compiler_params=pltpu.CompilerParams(
          needs_layout_passes=False, use_tc_tiling_on_sc=False),
      out_type=jax.ShapeDtypeStruct((B,), jnp.float32),
      scratch_types=[
          pltpu.VMEM((F, NB), jnp.int32),
          pltpu.VMEM((F, NB), jnp.float32),
          pltpu.VMEM((IPC,), jnp.float32),
          pltpu.VMEM((IPC, D), jnp.float32),
          pltpu.VMEM((F, NB), jnp.int32),
          pltpu.VMEM((F, NB), jnp.float32),
          pltpu.VMEM((IPC,), jnp.float32),
          pltpu.VMEM((IPC, D), jnp.float32),
          pltpu.VMEM((L,), jnp.float32),
          pltpu.VMEM((RPW,), jnp.float32),
          pltpu.SemaphoreType.DMA,
          pltpu.SemaphoreType.DMA,
          pltpu.SemaphoreType.DMA,
          pltpu.SemaphoreType.DMA,
      ],
  )
  def fm(fi_hbm, fv_hbm, emb_hbm, fo_hbm, bias_hbm, out_hbm,
         idx_a, fv_a, fo_a, rows_a, idx_b, fv_b, fo_b, rows_b,
         bias_v, out_v, sem_ra, sem_fa, sem_rb, sem_fb):
    wid = lax.axis_index("s") * NC + lax.axis_index("c")
    pltpu.sync_copy(bias_hbm, bias_v)
    iota = lax.iota(jnp.int32, L)

    def fire(c, idx_v, fv_v, fo_v, rows_v, sem_r, sem_f):
      base = wid * RPW + c * NB
      pltpu.sync_copy(fi_hbm.at[:, pl.ds(base, NB)], idx_v)
      pltpu.sync_copy(fv_hbm.at[:, pl.ds(base, NB)], fv_v)
      for f in range(F):
        pltpu.async_copy(
            emb_hbm.at[idx_v.at[f]], rows_v.at[pl.ds(f * NB, NB)], sem_r)
        pltpu.async_copy(
            fo_hbm.at[idx_v.at[f]], fo_v.at[pl.ds(f * NB, NB)], sem_f)

    def drain(idx_v, fo_v, rows_v, sem_r, sem_f):
      for f in range(F):
        pltpu.make_async_copy(
            emb_hbm.at[idx_v.at[f]], rows_v.at[pl.ds(f * NB, NB)],
            sem_r).wait()
        pltpu.make_async_copy(
            fo_hbm.at[idx_v.at[f]], fo_v.at[pl.ds(f * NB, NB)],
            sem_f).wait()

    dsel = [(iota + k) % D for k in range(D)]

    def compute(c, fv_v, fo_v, rows_v):
      def grp_body(g, gcarry):
        g16 = g * L
        zero = jnp.zeros((L,), jnp.float32)
        acc1 = zero
        acc_q = [zero] * 4
        acc_s = [zero] * D
        for f in range(F):
          rid = iota + (f * NB + g16)
          fvv = fv_v[f, pl.ds(g16, L)]
          wv = fo_v[pl.ds(f * NB + g16, L)]
          acc1 = acc1 + fvv * wv
          for k in range(D):
            e = plsc.load_gather(rows_v, [rid, dsel[k]])
            x = fvv * e
            acc_s[k] = acc_s[k] + x
            acc_q[k % 4] = acc_q[k % 4] + x * x
        s2 = jnp.zeros((L,), jnp.float32)
        for k in range(D):
          s2 = s2 + acc_s[k] * acc_s[k]
        sq = (acc_q[0] + acc_q[1]) + (acc_q[2] + acc_q[3])
        t = bias_v[...] + acc1 + 0.5 * (s2 - sq)
        y = 1.0 / (1.0 + jnp.exp(-t))
        out_v[pl.ds(c * NB + g16, L)] = y
        return gcarry

      lax.fori_loop(0, NGRP, grp_body, 0)

    fire(0, idx_a, fv_a, fo_a, rows_a, sem_ra, sem_fa)

    def pair_body(k, carry):
      c0 = 2 * k
      fire(c0 + 1, idx_b, fv_b, fo_b, rows_b, sem_rb, sem_fb)
      drain(idx_a, fo_a, rows_a, sem_ra, sem_fa)
      compute(c0, fv_a, fo_a, rows_a)

      @pl.when(k < NCHUNK // 2 - 1)
      def _():
        fire(c0 + 2, idx_a, fv_a, fo_a, rows_a, sem_ra, sem_fa)

      drain(idx_b, fo_b, rows_b, sem_rb, sem_fb)
      compute(c0 + 1, fv_b, fo_b, rows_b)
      return carry

    lax.fori_loop(0, NCHUNK // 2, pair_body, 0)
    pltpu.sync_copy(out_v, out_hbm.at[pl.ds(wid * RPW, RPW)])

  return fm


def kernel(feat_index, feat_value, emb_table, first_order_w, bias):
  B, F = feat_index.shape
  V, D = emb_table.shape
  fi_t = feat_index.astype(jnp.int32).T
  fv_t = feat_value.astype(jnp.float32).T
  emb_t = emb_table.astype(jnp.float32).T
  fo = first_order_w.astype(jnp.float32).T.reshape(V)
  bias16 = jnp.broadcast_to(bias.astype(jnp.float32), (16,))
  emb_lin = _build_relayout(V, D)(emb_t).reshape(V, D)
  fm = _build_fm(B, F, V, D)
  return fm(fi_t, fv_t, emb_lin, fo, bias16)
